# speedup vs baseline: 1.4410x; 1.4410x over previous
import jax
import jax.numpy as jnp
from jax import lax
from jax.experimental import pallas as pl
from jax.experimental.pallas import tpu as pltpu
from jax.experimental.pallas import tpu_sc as plsc

NUM_CORES = 2
NUM_SUBCORES = 16
NW = NUM_CORES * NUM_SUBCORES

VOCAB = 100000
D = 1024
N_TOK = 4 * 4096

B_PER_W = N_TOK // NW
CHUNK = 32
N_CHUNK = B_PER_W // CHUNK


def _emb_kernel(table_hbm, ids_hbm, out_hbm, idx_v, rows_v, sem):
    wid = lax.axis_index("s") * NUM_CORES + lax.axis_index("c")
    base = wid * B_PER_W
    pltpu.sync_copy(ids_hbm.at[pl.ds(base, B_PER_W)], idx_v)

    def body(j, carry):
        pltpu.async_copy(
            table_hbm.at[idx_v.at[pl.ds(j * CHUNK, CHUNK)]], rows_v, sem
        ).wait()
        pltpu.sync_copy(rows_v, out_hbm.at[pl.ds(base + j * CHUNK, CHUNK)])
        return carry

    lax.fori_loop(0, N_CHUNK, body, 0)


@jax.jit
def _embed(ids_flat, table):
    kfn = pl.kernel(
        _emb_kernel,
        out_type=jax.ShapeDtypeStruct((N_TOK, D), jnp.float32),
        mesh=plsc.VectorSubcoreMesh(
            core_axis_name="c", subcore_axis_name="s",
            num_cores=NUM_CORES, num_subcores=NUM_SUBCORES,
        ),
        scratch_types=[
            pltpu.VMEM((B_PER_W,), jnp.int32),
            pltpu.VMEM((CHUNK, D), jnp.float32),
            pltpu.SemaphoreType.DMA,
        ],
    )
    return kfn(table, ids_flat)


def kernel(token_ids, table):
    ids_flat = token_ids.reshape(-1).astype(jnp.int32)
    out = _embed(ids_flat, table)
    return out.reshape(token_ids.shape + (D,))


# speedup vs baseline: 1.6331x; 1.1334x over previous
import jax
import jax.numpy as jnp
from jax import lax
from jax.experimental import pallas as pl
from jax.experimental.pallas import tpu as pltpu
from jax.experimental.pallas import tpu_sc as plsc

NUM_CORES = 2
NUM_SUBCORES = 16
NW = NUM_CORES * NUM_SUBCORES

VOCAB = 100000
D = 1024
N_TOK = 4 * 4096

B_PER_W = N_TOK // NW
CHUNK = 32
N_CHUNK = B_PER_W // CHUNK


NBUF = 2


def _emb_kernel(table_hbm, ids_hbm, out_hbm, idx_v, rows, gsems, ssems):
    wid = lax.axis_index("s") * NUM_CORES + lax.axis_index("c")
    base = wid * B_PER_W
    pltpu.sync_copy(ids_hbm.at[pl.ds(base, B_PER_W)], idx_v)

    def gather(j):
        b = j % NBUF
        return pltpu.async_copy(
            table_hbm.at[idx_v.at[pl.ds(j * CHUNK, CHUNK)]], rows[b], gsems[b]
        )

    def store(j):
        b = j % NBUF
        return pltpu.async_copy(
            rows[b], out_hbm.at[pl.ds(base + j * CHUNK, CHUNK)], ssems[b]
        )

    g_desc = [None] * N_CHUNK
    s_desc = [None] * N_CHUNK
    for j in range(NBUF):
        g_desc[j] = gather(j)
    for j in range(N_CHUNK):
        g_desc[j].wait()
        s_desc[j] = store(j)
        if j + NBUF < N_CHUNK:
            s_desc[j].wait()
            g_desc[j + NBUF] = gather(j + NBUF)
    for j in range(N_CHUNK - NBUF, N_CHUNK):
        s_desc[j].wait()


@jax.jit
def _embed(ids_flat, table):
    kfn = pl.kernel(
        _emb_kernel,
        out_type=jax.ShapeDtypeStruct((N_TOK, D), jnp.float32),
        mesh=plsc.VectorSubcoreMesh(
            core_axis_name="c", subcore_axis_name="s",
            num_cores=NUM_CORES, num_subcores=NUM_SUBCORES,
        ),
        scratch_types=[
            pltpu.VMEM((B_PER_W,), jnp.int32),
            [pltpu.VMEM((CHUNK, D), jnp.float32) for _ in range(NBUF)],
            [pltpu.SemaphoreType.DMA for _ in range(NBUF)],
            [pltpu.SemaphoreType.DMA for _ in range(NBUF)],
        ],
    )
    return kfn(table, ids_flat)


def kernel(token_ids, table):
    ids_flat = token_ids.reshape(-1).astype(jnp.int32)
    out = _embed(ids_flat, table)
    return out.reshape(token_ids.shape + (D,))


# speedup vs baseline: 1.6546x; 1.0132x over previous
import jax
import jax.numpy as jnp
from jax import lax
from jax.experimental import pallas as pl
from jax.experimental.pallas import tpu as pltpu
from jax.experimental.pallas import tpu_sc as plsc

NUM_CORES = 2
NUM_SUBCORES = 16
NW = NUM_CORES * NUM_SUBCORES

VOCAB = 100000
D = 1024
N_TOK = 4 * 4096

B_PER_W = N_TOK // NW
CHUNK = 32
N_CHUNK = B_PER_W // CHUNK


NBUF = 3


def _emb_kernel(table_hbm, ids_hbm, out_hbm, idx_v, rows, gsems, ssems):
    wid = lax.axis_index("s") * NUM_CORES + lax.axis_index("c")
    base = wid * B_PER_W
    pltpu.sync_copy(ids_hbm.at[pl.ds(base, B_PER_W)], idx_v)

    def gather(j):
        b = j % NBUF
        return pltpu.async_copy(
            table_hbm.at[idx_v.at[pl.ds(j * CHUNK, CHUNK)]], rows[b], gsems[b]
        )

    def store(j):
        b = j % NBUF
        return pltpu.async_copy(
            rows[b], out_hbm.at[pl.ds(base + j * CHUNK, CHUNK)], ssems[b]
        )

    g_desc = [None] * N_CHUNK
    s_desc = [None] * N_CHUNK
    for j in range(NBUF):
        g_desc[j] = gather(j)
    for j in range(N_CHUNK):
        g_desc[j].wait()
        s_desc[j] = store(j)
        if j + NBUF < N_CHUNK:
            s_desc[j].wait()
            g_desc[j + NBUF] = gather(j + NBUF)
    for j in range(N_CHUNK - NBUF, N_CHUNK):
        s_desc[j].wait()


@jax.jit
def _embed(ids_flat, table):
    kfn = pl.kernel(
        _emb_kernel,
        out_type=jax.ShapeDtypeStruct((N_TOK, D), jnp.float32),
        mesh=plsc.VectorSubcoreMesh(
            core_axis_name="c", subcore_axis_name="s",
            num_cores=NUM_CORES, num_subcores=NUM_SUBCORES,
        ),
        scratch_types=[
            pltpu.VMEM((B_PER_W,), jnp.int32),
            [pltpu.VMEM((CHUNK, D), jnp.float32) for _ in range(NBUF)],
            [pltpu.SemaphoreType.DMA for _ in range(NBUF)],
            [pltpu.SemaphoreType.DMA for _ in range(NBUF)],
        ],
    )
    return kfn(table, ids_flat)


def kernel(token_ids, table):
    ids_flat = token_ids.reshape(-1).astype(jnp.int32)
    out = _embed(ids_flat, table)
    return out.reshape(token_ids.shape + (D,))


# speedup vs baseline: 1.6657x; 1.0067x over previous
import jax
import jax.numpy as jnp
from jax import lax
from jax.experimental import pallas as pl
from jax.experimental.pallas import tpu as pltpu
from jax.experimental.pallas import tpu_sc as plsc

NUM_CORES = 2
NUM_SUBCORES = 16
NW = NUM_CORES * NUM_SUBCORES

VOCAB = 100000
D = 1024
N_TOK = 4 * 4096

B_PER_W = N_TOK // NW
CHUNK = 16
N_CHUNK = B_PER_W // CHUNK


NBUF = 6


def _emb_kernel(table_hbm, ids_hbm, out_hbm, idx_v, rows, gsems, ssems):
    wid = lax.axis_index("s") * NUM_CORES + lax.axis_index("c")
    base = wid * B_PER_W
    pltpu.sync_copy(ids_hbm.at[pl.ds(base, B_PER_W)], idx_v)

    def gather(j):
        b = j % NBUF
        return pltpu.async_copy(
            table_hbm.at[idx_v.at[pl.ds(j * CHUNK, CHUNK)]], rows[b], gsems[b]
        )

    def store(j):
        b = j % NBUF
        return pltpu.async_copy(
            rows[b], out_hbm.at[pl.ds(base + j * CHUNK, CHUNK)], ssems[b]
        )

    g_desc = [None] * N_CHUNK
    s_desc = [None] * N_CHUNK
    for j in range(NBUF):
        g_desc[j] = gather(j)
    for j in range(N_CHUNK):
        g_desc[j].wait()
        s_desc[j] = store(j)
        if j + NBUF < N_CHUNK:
            s_desc[j].wait()
            g_desc[j + NBUF] = gather(j + NBUF)
    for j in range(N_CHUNK - NBUF, N_CHUNK):
        s_desc[j].wait()


@jax.jit
def _embed(ids_flat, table):
    kfn = pl.kernel(
        _emb_kernel,
        out_type=jax.ShapeDtypeStruct((N_TOK, D), jnp.float32),
        mesh=plsc.VectorSubcoreMesh(
            core_axis_name="c", subcore_axis_name="s",
            num_cores=NUM_CORES, num_subcores=NUM_SUBCORES,
        ),
        scratch_types=[
            pltpu.VMEM((B_PER_W,), jnp.int32),
            [pltpu.VMEM((CHUNK, D), jnp.float32) for _ in range(NBUF)],
            [pltpu.SemaphoreType.DMA for _ in range(NBUF)],
            [pltpu.SemaphoreType.DMA for _ in range(NBUF)],
        ],
    )
    return kfn(table, ids_flat)


def kernel(token_ids, table):
    ids_flat = token_ids.reshape(-1).astype(jnp.int32)
    out = _embed(ids_flat, table)
    return out.reshape(token_ids.shape + (D,))
